# speedup vs baseline: 2.5581x; 2.5581x over previous
import functools
import math

import jax
import jax.numpy as jnp
from jax import lax
from jax.experimental import pallas as pl
from jax.experimental.pallas import tpu as pltpu
from jax.experimental.pallas import tpu_sc as plsc

D_MODEL = 128
SCALE = math.sqrt(D_MODEL)
NUM_CORES = 2
NUM_SUBCORES = 16
NUM_WORKERS = NUM_CORES * NUM_SUBCORES
LANES = 16
CHUNK = 256


def _emb_body(b_per_w, table_hbm, idx_hbm, out_hbm, idx_v, rows_v, sem):
    wid = lax.axis_index("s") * NUM_CORES + lax.axis_index("c")
    base = wid * b_per_w
    pltpu.sync_copy(idx_hbm.at[pl.ds(base, b_per_w)], idx_v)

    n_chunks = b_per_w // CHUNK

    @pl.loop(0, n_chunks)
    def _chunk(g):
        cbase = g * CHUNK
        pltpu.async_copy(
            table_hbm.at[idx_v.at[pl.ds(cbase, CHUNK)]], rows_v, sem
        ).wait()

        @pl.loop(0, CHUNK)
        def _scale(r):
            for c in range(D_MODEL // LANES):
                sl = pl.ds(c * LANES, LANES)
                rows_v[r, sl] = rows_v[r, sl] * SCALE

        pltpu.sync_copy(rows_v, out_hbm.at[pl.ds(base + cbase, CHUNK)])


@jax.jit
def kernel(x, emb_weight):
    batch, seq = x.shape
    b_total = batch * seq
    b_per_w = b_total // NUM_WORKERS
    x_flat = x.reshape(b_total).astype(jnp.int32)

    mesh = plsc.VectorSubcoreMesh(core_axis_name="c", subcore_axis_name="s")
    out_flat = pl.kernel(
        functools.partial(_emb_body, b_per_w),
        out_type=jax.ShapeDtypeStruct((b_total, D_MODEL), jnp.float32),
        mesh=mesh,
        scratch_types=[
            pltpu.VMEM((b_per_w,), jnp.int32),
            pltpu.VMEM((CHUNK, D_MODEL), jnp.float32),
            pltpu.SemaphoreType.DMA,
        ],
    )(emb_weight, x_flat)
    return out_flat.reshape(batch, seq, D_MODEL)


# speedup vs baseline: 2.9322x; 1.1463x over previous
import functools
import math

import jax
import jax.numpy as jnp
from jax import lax
from jax.experimental import pallas as pl
from jax.experimental.pallas import tpu as pltpu
from jax.experimental.pallas import tpu_sc as plsc

D_MODEL = 128
SCALE = math.sqrt(D_MODEL)
NUM_CORES = 2
NUM_SUBCORES = 16
NUM_WORKERS = NUM_CORES * NUM_SUBCORES
LANES = 16
CHUNK = 400


def _scale_chunk(buf):
    @pl.loop(0, CHUNK, unroll=4)
    def _scale(r):
        for c in range(D_MODEL // LANES):
            sl = pl.ds(c * LANES, LANES)
            buf[r, sl] = buf[r, sl] * SCALE


def _emb_body(b_per_w, table_hbm, idx_hbm, out_hbm, idx_v, buf0, buf1, sem0, sem1):
    wid = lax.axis_index("s") * NUM_CORES + lax.axis_index("c")
    base = wid * b_per_w
    pltpu.sync_copy(idx_hbm.at[pl.ds(base, b_per_w)], idx_v)

    n_chunks = b_per_w // CHUNK
    n_pairs = n_chunks // 2
    bufs = (buf0, buf1)
    sems = (sem0, sem1)

    def start_gather(g, b):
        pltpu.async_copy(
            table_hbm.at[idx_v.at[pl.ds(g * CHUNK, CHUNK)]], bufs[b], sems[b]
        )

    def wait_gather(b):
        pltpu.make_async_copy(table_hbm.at[pl.ds(0, CHUNK)], bufs[b], sems[b]).wait()

    def write_out(g, b):
        pltpu.sync_copy(bufs[b], out_hbm.at[pl.ds(base + g * CHUNK, CHUNK)])

    start_gather(0, 0)

    @pl.loop(0, n_pairs)
    def _pair(p):
        g0 = 2 * p
        wait_gather(0)
        start_gather(g0 + 1, 1)
        _scale_chunk(buf0)
        write_out(g0, 0)

        wait_gather(1)

        @pl.when(p < n_pairs - 1)
        def _():
            start_gather(g0 + 2, 0)

        _scale_chunk(buf1)
        write_out(g0 + 1, 1)


@jax.jit
def kernel(x, emb_weight):
    batch, seq = x.shape
    b_total = batch * seq
    b_per_w = b_total // NUM_WORKERS
    x_flat = x.reshape(b_total).astype(jnp.int32)

    mesh = plsc.VectorSubcoreMesh(core_axis_name="c", subcore_axis_name="s")
    out_flat = pl.kernel(
        functools.partial(_emb_body, b_per_w),
        out_type=jax.ShapeDtypeStruct((b_total, D_MODEL), jnp.float32),
        mesh=mesh,
        scratch_types=[
            pltpu.VMEM((b_per_w,), jnp.int32),
            pltpu.VMEM((CHUNK, D_MODEL), jnp.float32),
            pltpu.VMEM((CHUNK, D_MODEL), jnp.float32),
            pltpu.SemaphoreType.DMA,
            pltpu.SemaphoreType.DMA,
        ],
    )(emb_weight, x_flat)
    return out_flat.reshape(batch, seq, D_MODEL)


# speedup vs baseline: 2.9361x; 1.0013x over previous
import functools
import math

import jax
import jax.numpy as jnp
from jax import lax
from jax.experimental import pallas as pl
from jax.experimental.pallas import tpu as pltpu
from jax.experimental.pallas import tpu_sc as plsc

D_MODEL = 128
SCALE = math.sqrt(D_MODEL)
NUM_CORES = 2
NUM_SUBCORES = 16
NUM_WORKERS = NUM_CORES * NUM_SUBCORES
LANES = 16
CHUNK = 400


def _scale_chunk(buf):
    @plsc.parallel_loop(0, CHUNK, unroll=4)
    def _scale(r):
        for c in range(D_MODEL // LANES):
            sl = pl.ds(c * LANES, LANES)
            buf[r, sl] = buf[r, sl] * SCALE


def _emb_body(
    b_per_w, table_hbm, idx_hbm, out_hbm, idx_v, buf0, buf1, gsem0, gsem1, osem0, osem1
):
    wid = lax.axis_index("s") * NUM_CORES + lax.axis_index("c")
    base = wid * b_per_w
    pltpu.sync_copy(idx_hbm.at[pl.ds(base, b_per_w)], idx_v)

    n_chunks = b_per_w // CHUNK
    n_pairs = n_chunks // 2
    bufs = (buf0, buf1)
    gsems = (gsem0, gsem1)
    osems = (osem0, osem1)

    def start_gather(g, b):
        pltpu.async_copy(
            table_hbm.at[idx_v.at[pl.ds(g * CHUNK, CHUNK)]], bufs[b], gsems[b]
        )

    def wait_gather(b):
        pltpu.make_async_copy(table_hbm.at[pl.ds(0, CHUNK)], bufs[b], gsems[b]).wait()

    def start_out(g, b):
        pltpu.async_copy(bufs[b], out_hbm.at[pl.ds(base + g * CHUNK, CHUNK)], osems[b])

    def wait_out(b):
        pltpu.make_async_copy(bufs[b], out_hbm.at[pl.ds(base, CHUNK)], osems[b]).wait()

    start_gather(0, 0)

    @pl.loop(0, n_pairs)
    def _pair(p):
        g0 = 2 * p
        wait_gather(0)

        @pl.when(p > 0)
        def _():
            wait_out(1)

        start_gather(g0 + 1, 1)
        _scale_chunk(buf0)
        start_out(g0, 0)

        wait_gather(1)
        wait_out(0)

        @pl.when(p < n_pairs - 1)
        def _():
            start_gather(g0 + 2, 0)

        _scale_chunk(buf1)
        start_out(g0 + 1, 1)

    wait_out(1)


@jax.jit
def kernel(x, emb_weight):
    batch, seq = x.shape
    b_total = batch * seq
    b_per_w = b_total // NUM_WORKERS
    x_flat = x.reshape(b_total).astype(jnp.int32)

    mesh = plsc.VectorSubcoreMesh(core_axis_name="c", subcore_axis_name="s")
    out_flat = pl.kernel(
        functools.partial(_emb_body, b_per_w),
        out_type=jax.ShapeDtypeStruct((b_total, D_MODEL), jnp.float32),
        mesh=mesh,
        scratch_types=[
            pltpu.VMEM((b_per_w,), jnp.int32),
            pltpu.VMEM((CHUNK, D_MODEL), jnp.float32),
            pltpu.VMEM((CHUNK, D_MODEL), jnp.float32),
            pltpu.SemaphoreType.DMA,
            pltpu.SemaphoreType.DMA,
            pltpu.SemaphoreType.DMA,
            pltpu.SemaphoreType.DMA,
        ],
    )(emb_weight, x_flat)
    return out_flat.reshape(batch, seq, D_MODEL)


# speedup vs baseline: 5.1848x; 1.7659x over previous
import functools
import math

import jax
import jax.numpy as jnp
from jax import lax
from jax.experimental import pallas as pl
from jax.experimental.pallas import tpu as pltpu
from jax.experimental.pallas import tpu_sc as plsc

D_MODEL = 128
SCALE = math.sqrt(D_MODEL)
NUM_CORES = 2
NUM_SUBCORES = 16
NUM_WORKERS = NUM_CORES * NUM_SUBCORES
LANES = 16
CHUNK = 400


def _scale_chunk(buf):
    @plsc.parallel_loop(0, CHUNK, unroll=4)
    def _scale(r):
        for c in range(D_MODEL // LANES):
            sl = pl.ds(c * LANES, LANES)
            buf[r, sl] = buf[r, sl] * SCALE


def _emb_body(
    b_per_w, seq, table_hbm, idx_hbm, out_hbm, idx_v, buf0, buf1,
    gsem0, gsem1, osem0, osem1
):
    wid = lax.axis_index("s") * NUM_CORES + lax.axis_index("c")
    base = wid * b_per_w
    pltpu.sync_copy(idx_hbm.at[pl.ds(base, b_per_w)], idx_v)

    n_chunks = b_per_w // CHUNK
    n_pairs = n_chunks // 2
    rows_per_chunk = CHUNK // seq
    bufs = (buf0, buf1)
    gsems = (gsem0, gsem1)
    osems = (osem0, osem1)

    def start_gather(g, b):
        pltpu.async_copy(
            table_hbm.at[idx_v.at[pl.ds(g * CHUNK, CHUNK)]], bufs[b], gsems[b]
        )

    def wait_gather(b):
        pltpu.make_async_copy(table_hbm.at[pl.ds(0, CHUNK)], bufs[b], gsems[b]).wait()

    def start_out(g, b):
        bbase = (base + g * CHUNK) // seq
        for i in range(rows_per_chunk):
            pltpu.async_copy(
                bufs[b].at[pl.ds(i * seq, seq)], out_hbm.at[bbase + i], osems[b]
            )

    def wait_out(b):
        pltpu.make_async_copy(
            bufs[b], out_hbm.at[pl.ds(0, rows_per_chunk)], osems[b]
        ).wait()

    start_gather(0, 0)

    @pl.loop(0, n_pairs)
    def _pair(p):
        g0 = 2 * p
        wait_gather(0)

        @pl.when(p > 0)
        def _():
            wait_out(1)

        start_gather(g0 + 1, 1)
        _scale_chunk(buf0)
        start_out(g0, 0)

        wait_gather(1)
        wait_out(0)

        @pl.when(p < n_pairs - 1)
        def _():
            start_gather(g0 + 2, 0)

        _scale_chunk(buf1)
        start_out(g0 + 1, 1)

    wait_out(1)


@jax.jit
def kernel(x, emb_weight):
    batch, seq = x.shape
    b_total = batch * seq
    b_per_w = b_total // NUM_WORKERS
    x_flat = x.reshape(b_total).astype(jnp.int32)

    mesh = plsc.VectorSubcoreMesh(core_axis_name="c", subcore_axis_name="s")
    out = pl.kernel(
        functools.partial(_emb_body, b_per_w, seq),
        out_type=jax.ShapeDtypeStruct((batch, seq, D_MODEL), jnp.float32),
        mesh=mesh,
        scratch_types=[
            pltpu.VMEM((b_per_w,), jnp.int32),
            pltpu.VMEM((CHUNK, D_MODEL), jnp.float32),
            pltpu.VMEM((CHUNK, D_MODEL), jnp.float32),
            pltpu.SemaphoreType.DMA,
            pltpu.SemaphoreType.DMA,
            pltpu.SemaphoreType.DMA,
            pltpu.SemaphoreType.DMA,
        ],
    )(emb_weight, x_flat)
    return out
